# Initial kernel scaffold; baseline (speedup 1.0000x reference)
import jax
import jax.numpy as jnp
from jax.experimental import pallas as pl


def kernel(x, edge_index, Wl, bl, Wr, br, att, bias, gamma, beta):
    raise NotImplementedError("write your pallas kernel here")


# speedup vs baseline: 42.0228x; 42.0228x over previous
import functools

import jax
import jax.numpy as jnp
from jax import lax
from jax.experimental import pallas as pl
from jax.experimental.pallas import tpu as pltpu
from jax.experimental.pallas import tpu_sc as plsc

_N = 10000
_D = 128
_HC = 128
_NH = 4
_EN = 330000

_NW = 32
_CHUNK = 64
_CPW = 162
_EPAD = _NW * _CPW * _CHUNK
_NACC = 10240
_RPT = _NACC // 16
_ACCW = 144


def _mm_body(xb, wl, wr, blb, brb, xl_out, xr_out):
    xv = xb[...]
    xl_out[...] = jnp.dot(xv, wl[...], preferred_element_type=jnp.float32) + blb[...]
    xr_out[...] = jnp.dot(xv, wr[...], preferred_element_type=jnp.float32) + brb[...]


def _mm_call(x, Wl, Wr, bl, br):
    bn = 1000
    grid = (_N // bn,)
    return pl.pallas_call(
        _mm_body,
        grid=grid,
        in_specs=[
            pl.BlockSpec((bn, _D), lambda i: (i, 0)),
            pl.BlockSpec((_D, _HC), lambda i: (0, 0)),
            pl.BlockSpec((_D, _HC), lambda i: (0, 0)),
            pl.BlockSpec((1, _HC), lambda i: (0, 0)),
            pl.BlockSpec((1, _HC), lambda i: (0, 0)),
        ],
        out_specs=[
            pl.BlockSpec((bn, _HC), lambda i: (i, 0)),
            pl.BlockSpec((bn, _HC), lambda i: (i, 0)),
        ],
        out_shape=[
            jax.ShapeDtypeStruct((_N, _HC), jnp.float32),
            jax.ShapeDtypeStruct((_N, _HC), jnp.float32),
        ],
    )(x, Wl, Wr, bl.reshape(1, _HC), br.reshape(1, _HC))


def _sc_edge_body(xl_hbm, xr_hbm, src_hbm, dst_hbm, att_hbm, acc_out,
                  src_idx, dst_idx, xl_rows, xr_rows, msg, attv, acc_sh,
                  sem1, sem2):
    c = lax.axis_index("c")
    s = lax.axis_index("s")
    w = c * 16 + s

    pltpu.sync_copy(att_hbm, attv)

    zv = jnp.zeros((16,), jnp.float32)

    def _zero_row(e, carry):
        for k in range(_ACCW // 16):
            msg[e, pl.ds(16 * k, 16)] = zv
        return carry

    lax.fori_loop(0, _CHUNK, _zero_row, 0)

    def _zero_acc(b, carry):
        pltpu.sync_copy(msg, acc_sh.at[pl.ds(s * _RPT + b * _CHUNK, _CHUNK)])
        return carry

    lax.fori_loop(0, _RPT // _CHUNK, _zero_acc, 0)
    plsc.subcore_barrier()

    att_v = [attv[pl.ds(16 * j, 16)] for j in range(8)]
    lane = lax.iota(jnp.int32, 16)
    x1 = jnp.bitwise_xor(lane, 1)
    x2 = jnp.bitwise_xor(lane, 2)
    x4 = jnp.bitwise_xor(lane, 4)
    x8 = jnp.bitwise_xor(lane, 8)
    even = jnp.bitwise_and(lane, 1) == 0
    low2 = lane < 2

    def _edge(e, carry):
        xlv = [xl_rows[e, pl.ds(16 * j, 16)] for j in range(8)]
        xrv = [xr_rows[e, pl.ds(16 * j, 16)] for j in range(8)]
        t = []
        for j in range(8):
            v = xlv[j] + xrv[j]
            v = jnp.where(v >= 0.0, v, v * 0.2)
            t.append(v * att_v[j])
        u = [t[2 * h] + t[2 * h + 1] for h in range(_NH)]
        a = [u[h] + jnp.take(u[h], x1) for h in range(_NH)]
        m0 = jnp.where(even, a[0], jnp.take(a[1], x1))
        m1 = jnp.where(even, a[2], jnp.take(a[3], x1))
        for xk in (x2, x4, x8):
            m0 = m0 + jnp.take(m0, xk)
            m1 = m1 + jnp.take(m1, xk)
        lv = jnp.where(low2, m0, m1)
        pv = jnp.exp(lv)
        msg[e, pl.ds(_HC, 16)] = pv
        p = [pv[h] for h in range(_NH)]
        for j in range(8):
            msg[e, pl.ds(16 * j, 16)] = xlv[j] * p[j // 2]
        return carry

    def _chunk(b, carry):
        base = (w * _CPW + b) * _CHUNK
        pltpu.sync_copy(src_hbm.at[pl.ds(base, _CHUNK)], src_idx)
        pltpu.sync_copy(dst_hbm.at[pl.ds(base, _CHUNK)], dst_idx)
        cp1 = pltpu.async_copy(xl_hbm.at[src_idx], xl_rows, sem1)
        cp2 = pltpu.async_copy(xr_hbm.at[dst_idx], xr_rows, sem2)
        cp1.wait()
        cp2.wait()
        lax.fori_loop(0, _CHUNK, _edge, 0)
        pltpu.sync_copy(msg, acc_sh.at[dst_idx], add=True)
        return carry

    lax.fori_loop(0, _CPW, _chunk, 0)
    plsc.subcore_barrier()

    def _flush(b, carry):
        r0 = s * _RPT + b * _CHUNK
        pltpu.sync_copy(acc_sh.at[pl.ds(r0, _CHUNK)],
                        acc_out.at[c, pl.ds(r0, _CHUNK)])
        return carry

    lax.fori_loop(0, _RPT // _CHUNK, _flush, 0)


@functools.lru_cache(maxsize=1)
def _sc_edge():
    return pl.kernel(
        _sc_edge_body,
        out_type=jax.ShapeDtypeStruct((2, _NACC, _ACCW), jnp.float32),
        mesh=plsc.VectorSubcoreMesh(core_axis_name="c", subcore_axis_name="s",
                                    num_cores=2, num_subcores=16),
        scratch_types=[
            pltpu.VMEM((_CHUNK,), jnp.int32),
            pltpu.VMEM((_CHUNK,), jnp.int32),
            pltpu.VMEM((_CHUNK, _D), jnp.float32),
            pltpu.VMEM((_CHUNK, _D), jnp.float32),
            pltpu.VMEM((_CHUNK, _ACCW), jnp.float32),
            pltpu.VMEM((_HC,), jnp.float32),
            pltpu.VMEM_SHARED((_NACC, _ACCW), jnp.float32),
            pltpu.SemaphoreType.DMA,
            pltpu.SemaphoreType.DMA,
        ],
        compiler_params=pltpu.CompilerParams(use_tc_tiling_on_sc=False),
    )


def _fin_body(accb, xb, biasb, gammab, betab, ob):
    a = accb[0] + accb[1]
    num = a[:, :_HC]
    den = a[:, _HC:_HC + _NH] + 1e-16
    bn = num.shape[0]
    dv = jnp.concatenate(
        [jnp.broadcast_to(den[:, h:h + 1], (bn, 32)) for h in range(_NH)],
        axis=1)
    y = num / dv + biasb[...] + xb[...]
    mu = jnp.mean(y, axis=1, keepdims=True)
    var = jnp.mean((y - mu) ** 2, axis=1, keepdims=True)
    yn = (y - mu) * lax.rsqrt(var + 1e-5) * gammab[...] + betab[...]
    ob[...] = jnp.where(yn > 0.0, yn, jnp.exp(yn) - 1.0)


def _fin_call(acc, x, bias, gamma, beta):
    bn = 1000
    grid = (_N // bn,)
    return pl.pallas_call(
        _fin_body,
        grid=grid,
        in_specs=[
            pl.BlockSpec((2, bn, _ACCW), lambda i: (0, i, 0)),
            pl.BlockSpec((bn, _D), lambda i: (i, 0)),
            pl.BlockSpec((1, _HC), lambda i: (0, 0)),
            pl.BlockSpec((1, _HC), lambda i: (0, 0)),
            pl.BlockSpec((1, _HC), lambda i: (0, 0)),
        ],
        out_specs=pl.BlockSpec((bn, _HC), lambda i: (i, 0)),
        out_shape=jax.ShapeDtypeStruct((_N, _HC), jnp.float32),
    )(acc, x, bias.reshape(1, _HC), gamma.reshape(1, _HC),
      beta.reshape(1, _HC))


def kernel(x, edge_index, Wl, bl, Wr, br, att, bias, gamma, beta):
    loop = jnp.arange(_N, dtype=jnp.int32)
    src = jnp.concatenate([edge_index[0].astype(jnp.int32), loop])
    dst = jnp.concatenate([edge_index[1].astype(jnp.int32), loop])
    npad = _EPAD - _EN
    src = jnp.concatenate([src, jnp.zeros((npad,), jnp.int32)])
    dst = jnp.concatenate([dst, jnp.full((npad,), _N, jnp.int32)])

    xl, xr = _mm_call(x, Wl, Wr, bl, br)
    acc = _sc_edge()(xl, xr, src, dst, att.reshape(_HC))
    return _fin_call(acc, x, bias, gamma, beta)


# speedup vs baseline: 52.0921x; 1.2396x over previous
import functools

import jax
import jax.numpy as jnp
from jax import lax
from jax.experimental import pallas as pl
from jax.experimental.pallas import tpu as pltpu
from jax.experimental.pallas import tpu_sc as plsc

_N = 10000
_D = 128
_HC = 128
_NH = 4
_EN = 330000

_NW = 32
_CHUNK = 64
_CPW = 162
_EPAD = _NW * _CPW * _CHUNK
_NACC = 10240
_RPT = _NACC // 16
_ACCW = 144


def _mm_body(xb, wl, wr, blb, brb, xl_out, xr_out):
    xv = xb[...]
    xl_out[...] = jnp.dot(xv, wl[...], preferred_element_type=jnp.float32) + blb[...]
    xr_out[...] = jnp.dot(xv, wr[...], preferred_element_type=jnp.float32) + brb[...]


def _mm_call(x, Wl, Wr, bl, br):
    bn = 1000
    grid = (_N // bn,)
    return pl.pallas_call(
        _mm_body,
        grid=grid,
        in_specs=[
            pl.BlockSpec((bn, _D), lambda i: (i, 0)),
            pl.BlockSpec((_D, _HC), lambda i: (0, 0)),
            pl.BlockSpec((_D, _HC), lambda i: (0, 0)),
            pl.BlockSpec((1, _HC), lambda i: (0, 0)),
            pl.BlockSpec((1, _HC), lambda i: (0, 0)),
        ],
        out_specs=[
            pl.BlockSpec((bn, _HC), lambda i: (i, 0)),
            pl.BlockSpec((bn, _HC), lambda i: (i, 0)),
        ],
        out_shape=[
            jax.ShapeDtypeStruct((_N, _HC), jnp.float32),
            jax.ShapeDtypeStruct((_N, _HC), jnp.float32),
        ],
    )(x, Wl, Wr, bl.reshape(1, _HC), br.reshape(1, _HC))


def _sc_edge_body(xl_hbm, xr_hbm, src_hbm, dst_hbm, att_hbm, acc_out,
                  src_idx, dst_idx, xl_rows, xr_rows, msg, attv, acc_sh,
                  sem1, sem2):
    c = lax.axis_index("c")
    s = lax.axis_index("s")
    w = c * 16 + s

    pltpu.sync_copy(att_hbm, attv)

    zv = jnp.zeros((16,), jnp.float32)

    def _zero_row(e, carry):
        for k in range(_ACCW // 16):
            msg[e, pl.ds(16 * k, 16)] = zv
        return carry

    lax.fori_loop(0, _CHUNK, _zero_row, 0)

    def _zero_acc(b, carry):
        pltpu.sync_copy(msg, acc_sh.at[pl.ds(s * _RPT + b * _CHUNK, _CHUNK)])
        return carry

    lax.fori_loop(0, _RPT // _CHUNK, _zero_acc, 0)
    plsc.subcore_barrier()

    att_v = [attv[pl.ds(16 * j, 16)] for j in range(8)]
    lane = lax.iota(jnp.int32, 16)
    x1 = jnp.bitwise_xor(lane, 1)
    x2 = jnp.bitwise_xor(lane, 2)
    x4 = jnp.bitwise_xor(lane, 4)
    x8 = jnp.bitwise_xor(lane, 8)
    even = jnp.bitwise_and(lane, 1) == 0
    low2 = lane < 2

    def _edge(e):
        xlv = [xl_rows[e, pl.ds(16 * j, 16)] for j in range(8)]
        xrv = [xr_rows[e, pl.ds(16 * j, 16)] for j in range(8)]
        t = []
        for j in range(8):
            v = xlv[j] + xrv[j]
            v = jnp.where(v >= 0.0, v, v * 0.2)
            t.append(v * att_v[j])
        u = [t[2 * h] + t[2 * h + 1] for h in range(_NH)]
        a = [u[h] + jnp.take(u[h], x1) for h in range(_NH)]
        m0 = jnp.where(even, a[0], jnp.take(a[1], x1))
        m1 = jnp.where(even, a[2], jnp.take(a[3], x1))
        for xk in (x2, x4, x8):
            m0 = m0 + jnp.take(m0, xk)
            m1 = m1 + jnp.take(m1, xk)
        lv = jnp.where(low2, m0, m1)
        pv = jnp.exp(lv)
        msg[e, pl.ds(_HC, 16)] = pv
        p = [pv[h] for h in range(_NH)]
        for j in range(8):
            msg[e, pl.ds(16 * j, 16)] = xlv[j] * p[j // 2]

    def _chunk(b, carry):
        base = (w * _CPW + b) * _CHUNK
        pltpu.sync_copy(src_hbm.at[pl.ds(base, _CHUNK)], src_idx)
        pltpu.sync_copy(dst_hbm.at[pl.ds(base, _CHUNK)], dst_idx)
        cp1 = pltpu.async_copy(xl_hbm.at[src_idx], xl_rows, sem1)
        cp2 = pltpu.async_copy(xr_hbm.at[dst_idx], xr_rows, sem2)
        cp1.wait()
        cp2.wait()
        plsc.parallel_loop(0, _CHUNK, unroll=4)(_edge)
        pltpu.sync_copy(msg, acc_sh.at[dst_idx], add=True)
        return carry

    lax.fori_loop(0, _CPW, _chunk, 0)
    plsc.subcore_barrier()

    def _flush(b, carry):
        r0 = s * _RPT + b * _CHUNK
        pltpu.sync_copy(acc_sh.at[pl.ds(r0, _CHUNK)],
                        acc_out.at[c, pl.ds(r0, _CHUNK)])
        return carry

    lax.fori_loop(0, _RPT // _CHUNK, _flush, 0)


@functools.lru_cache(maxsize=1)
def _sc_edge():
    return pl.kernel(
        _sc_edge_body,
        out_type=jax.ShapeDtypeStruct((2, _NACC, _ACCW), jnp.float32),
        mesh=plsc.VectorSubcoreMesh(core_axis_name="c", subcore_axis_name="s",
                                    num_cores=2, num_subcores=16),
        scratch_types=[
            pltpu.VMEM((_CHUNK,), jnp.int32),
            pltpu.VMEM((_CHUNK,), jnp.int32),
            pltpu.VMEM((_CHUNK, _D), jnp.float32),
            pltpu.VMEM((_CHUNK, _D), jnp.float32),
            pltpu.VMEM((_CHUNK, _ACCW), jnp.float32),
            pltpu.VMEM((_HC,), jnp.float32),
            pltpu.VMEM_SHARED((_NACC, _ACCW), jnp.float32),
            pltpu.SemaphoreType.DMA,
            pltpu.SemaphoreType.DMA,
        ],
        compiler_params=pltpu.CompilerParams(use_tc_tiling_on_sc=False),
    )


def _fin_body(accb, xb, biasb, gammab, betab, ob):
    a = accb[0] + accb[1]
    num = a[:, :_HC]
    den = a[:, _HC:_HC + _NH] + 1e-16
    bn = num.shape[0]
    dv = jnp.concatenate(
        [jnp.broadcast_to(den[:, h:h + 1], (bn, 32)) for h in range(_NH)],
        axis=1)
    y = num / dv + biasb[...] + xb[...]
    mu = jnp.mean(y, axis=1, keepdims=True)
    var = jnp.mean((y - mu) ** 2, axis=1, keepdims=True)
    yn = (y - mu) * lax.rsqrt(var + 1e-5) * gammab[...] + betab[...]
    ob[...] = jnp.where(yn > 0.0, yn, jnp.exp(yn) - 1.0)


def _fin_call(acc, x, bias, gamma, beta):
    bn = 1000
    grid = (_N // bn,)
    return pl.pallas_call(
        _fin_body,
        grid=grid,
        in_specs=[
            pl.BlockSpec((2, bn, _ACCW), lambda i: (0, i, 0)),
            pl.BlockSpec((bn, _D), lambda i: (i, 0)),
            pl.BlockSpec((1, _HC), lambda i: (0, 0)),
            pl.BlockSpec((1, _HC), lambda i: (0, 0)),
            pl.BlockSpec((1, _HC), lambda i: (0, 0)),
        ],
        out_specs=pl.BlockSpec((bn, _HC), lambda i: (i, 0)),
        out_shape=jax.ShapeDtypeStruct((_N, _HC), jnp.float32),
    )(acc, x, bias.reshape(1, _HC), gamma.reshape(1, _HC),
      beta.reshape(1, _HC))


def kernel(x, edge_index, Wl, bl, Wr, br, att, bias, gamma, beta):
    loop = jnp.arange(_N, dtype=jnp.int32)
    src = jnp.concatenate([edge_index[0].astype(jnp.int32), loop])
    dst = jnp.concatenate([edge_index[1].astype(jnp.int32), loop])
    npad = _EPAD - _EN
    src = jnp.concatenate([src, jnp.zeros((npad,), jnp.int32)])
    dst = jnp.concatenate([dst, jnp.full((npad,), _N, jnp.int32)])

    xl, xr = _mm_call(x, Wl, Wr, bl, br)
    acc = _sc_edge()(xl, xr, src, dst, att.reshape(_HC))
    return _fin_call(acc, x, bias, gamma, beta)
